# baseline (speedup 1.0000x reference)
import functools

import jax
import jax.numpy as jnp
from jax import lax
from jax.experimental import pallas as pl
from jax.experimental.pallas import tpu as pltpu
from jax.experimental.pallas import tpu_sc as plsc

N = 10000
E = 320000
E_LG = 640000
ND = 128
ED = 128

_EPS = 1e-5


def _k1_body(lgx_ref, w12t_ref, bl_ref, a_ref, b_ref):
    ab = jnp.dot(lgx_ref[...], w12t_ref[...], preferred_element_type=jnp.float32)
    a_ref[...] = ab[:, :ED]
    b_ref[...] = ab[:, ED:] + bl_ref[...]


def _k1(lg_x, w12t, b_line):
    blk = 512
    grid = E // blk
    return pl.pallas_call(
        _k1_body,
        grid=(grid,),
        in_specs=[
            pl.BlockSpec((blk, ED), lambda i: (i, 0)),
            pl.BlockSpec((ED, 2 * ED), lambda i: (0, 0)),
            pl.BlockSpec((1, ED), lambda i: (0, 0)),
        ],
        out_specs=[
            pl.BlockSpec((blk, ED), lambda i: (i, 0)),
            pl.BlockSpec((blk, ED), lambda i: (i, 0)),
        ],
        out_shape=[
            jax.ShapeDtypeStruct((E, ED), jnp.float32),
            jax.ShapeDtypeStruct((E, ED), jnp.float32),
        ],
    )(lg_x, w12t, b_line.reshape(1, ED))


def _k3_body(ea_ref, lgx_ref, sums_ref, cnt_ref, wgt_ref, bg_ref, g_ref, bt_ref,
             lgxn_ref, out_ea_ref, gated_ref):
    inv4 = 1.0 / jnp.maximum(cnt_ref[0], 1.0)
    invt = inv4.T
    agg = jnp.concatenate(
        [sums_ref[128 * q:128 * (q + 1), :] * invt[:, q:q + 1] for q in range(4)],
        axis=0)
    lgxn = lgx_ref[...] + agg
    lgxn_ref[...] = lgxn
    s = ea_ref[...] + lgxn
    m = jnp.mean(s, axis=1, keepdims=True)
    v = jnp.mean((s - m) ** 2, axis=1, keepdims=True)
    ea = (s - m) / jnp.sqrt(v + _EPS) * g_ref[...] + bt_ref[...]
    gate = jax.nn.sigmoid(
        jnp.dot(ea, wgt_ref[...], preferred_element_type=jnp.float32) + bg_ref[...])
    out_ea_ref[...] = ea
    gated_ref[...] = gate * ea


def _k3(edge_attr, lg_x, sums, cnt2d, wgt, b_gate, g_edge, bt_edge):
    blk = 512
    grid = E // blk
    return pl.pallas_call(
        _k3_body,
        grid=(grid,),
        in_specs=[
            pl.BlockSpec((blk, ED), lambda i: (i, 0)),
            pl.BlockSpec((blk, ED), lambda i: (i, 0)),
            pl.BlockSpec((blk, ED), lambda i: (i, 0)),
            pl.BlockSpec((1, blk // 128, 128), lambda i: (i, 0, 0)),
            pl.BlockSpec((ED, ED), lambda i: (0, 0)),
            pl.BlockSpec((1, ED), lambda i: (0, 0)),
            pl.BlockSpec((1, ED), lambda i: (0, 0)),
            pl.BlockSpec((1, ED), lambda i: (0, 0)),
        ],
        out_specs=[
            pl.BlockSpec((blk, ED), lambda i: (i, 0)),
            pl.BlockSpec((blk, ED), lambda i: (i, 0)),
            pl.BlockSpec((blk, ED), lambda i: (i, 0)),
        ],
        out_shape=[
            jax.ShapeDtypeStruct((E, ED), jnp.float32),
            jax.ShapeDtypeStruct((E, ED), jnp.float32),
            jax.ShapeDtypeStruct((E, ED), jnp.float32),
        ],
    )(edge_attr, lg_x, sums, cnt2d, wgt, b_gate.reshape(1, ED),
      g_edge.reshape(1, ED), bt_edge.reshape(1, ED))


_SC_CORES = 2
_SC_TILES = 16
_SC_WORKERS = _SC_CORES * _SC_TILES


_K2_NB = 40
_K2_SEGS = 8064
_K2_CH = 5120
_K2_NCH = E_LG // _K2_CH
_K2_BATCH = 64


def _k2_body(dst_hbm, src_hbm, att_hbm, a_hbm, b_hbm, w3_hbm,
             sums_out, cnt_out,
             dst_ch, src_ch, att_ch, pos_c,
             sidx_a, gdst_a, didx_a, cval_a, arows_a, brows_a,
             sidx_b, gdst_b, didx_b, cval_b, arows_b, brows_b,
             w3_vm, zcnt_v, cfl_v,
             sums_sh, cnt_sh, semd, sema1, semb1, sema2, semb2):
    c = lax.axis_index("c")
    s = lax.axis_index("s")
    zp = _K2_SEGS // _SC_TILES

    pltpu.sync_copy(w3_hbm, w3_vm)
    w3v = [w3_vm[pl.ds(16 * v, 16)] for v in range(8)]

    zv16 = jnp.zeros((16,), jnp.float32)
    zv16i = jnp.zeros((16,), jnp.int32)

    def zc(i, carry):
        zcnt_v[pl.ds(16 * i, 16)] = zv16
        return carry

    lax.fori_loop(0, (zp + 15) // 16, zc, 0)

    def zg(i, carry):
        pos_c[pl.ds(16 * i, 16)] = zv16i
        return carry

    lax.fori_loop(0, (_K2_CH + 16) // 16, zg, 0)

    bufs0 = (sidx_a, gdst_a, didx_a, cval_a, arows_a, brows_a, sema1, sema2)
    bufs1 = (sidx_b, gdst_b, didx_b, cval_b, arows_b, brows_b, semb1, semb2)

    def per_bin(bi, carry):
        b = c * (_K2_NB // 2) + bi
        lo = b * _K2_SEGS

        def zr(i, carry2):
            for v in range(8):
                arows_a[i, pl.ds(16 * v, 16)] = zv16
            return carry2

        lax.fori_loop(0, _K2_BATCH, zr, 0)

        def z(i, carry2):
            r = (s + i * _SC_TILES) * _K2_BATCH
            pltpu.sync_copy(arows_a, sums_sh.at[pl.ds(r, _K2_BATCH)])
            return carry2

        lax.fori_loop(0, ((_K2_SEGS // _K2_BATCH) - s + _SC_TILES - 1) // _SC_TILES,
                      z, 0)
        pltpu.sync_copy(zcnt_v.at[pl.ds(0, zp)], cnt_sh.at[pl.ds(s * zp, zp)])
        plsc.subcore_barrier()

        def per_chunk(ic, carry2):
            cid = s + ic * _SC_TILES
            base_g = cid * _K2_CH
            cpd = pltpu.async_copy(dst_hbm.at[pl.ds(base_g, _K2_CH)], dst_ch, semd)
            cps = pltpu.async_copy(src_hbm.at[pl.ds(base_g, _K2_CH)], src_ch, sema1)
            cpa = pltpu.async_copy(att_hbm.at[pl.ds(base_g, _K2_CH)], att_ch, semb1)
            cpd.wait()
            cps.wait()
            cpa.wait()

            def sc(j, nacc):
                d = dst_ch[pl.ds(16 * j, 16)]
                m = (d >= lo) & (d < lo + _K2_SEGS)
                posv = 16 * j + lax.iota(jnp.int32, 16)
                plsc.store_compressed(pos_c.at[pl.ds(nacc, 16)], posv, mask=m)
                pc = plsc.all_reduce_population_count(m)
                return nacc + pc[0]

            nacc = lax.fori_loop(0, _K2_CH // 16, sc, 0)
            nb = (nacc + _K2_BATCH - 1) // _K2_BATCH

            def issue(k, bufs):
                sidx, gdst, didx, cval, arows, brows, s1, s2 = bufs
                kb = k * _K2_BATCH
                for v in range(_K2_BATCH // 16):
                    pos = pos_c[pl.ds(kb + 16 * v, 16)]
                    dv = plsc.load_gather(dst_ch, [pos])
                    sv = plsc.load_gather(src_ch, [pos])
                    valid = (kb + 16 * v + lax.iota(jnp.int32, 16)) < nacc
                    sidx[pl.ds(16 * v, 16)] = sv
                    gdst[pl.ds(16 * v, 16)] = dv
                    didx[pl.ds(16 * v, 16)] = jnp.where(valid, dv - lo, _K2_SEGS)
                    cval[pl.ds(16 * v, 16)] = jnp.where(valid, 1.0, 0.0)
                pltpu.async_copy(a_hbm.at[sidx], arows, s1)
                pltpu.async_copy(b_hbm.at[gdst], brows, s2)

            def finish(k, bufs):
                sidx, gdst, didx, cval, arows, brows, s1, s2 = bufs
                kb = k * _K2_BATCH
                pltpu.make_async_copy(a_hbm.at[sidx], arows, s1).wait()
                pltpu.make_async_copy(b_hbm.at[gdst], brows, s2).wait()

                def rowg(g, carry4):
                    attv = plsc.load_gather(att_ch, [pos_c[pl.ds(kb + 16 * g, 16)]])
                    for r2 in range(16):
                        att = attv[r2]
                        r = 16 * g + r2
                        for v in range(8):
                            av = arows[r, pl.ds(16 * v, 16)]
                            bv = brows[r, pl.ds(16 * v, 16)]
                            zv = av + bv + att * w3v[v]
                            arows[r, pl.ds(16 * v, 16)] = zv / (1.0 + jnp.exp(-zv))
                    return carry4

                lax.fori_loop(0, _K2_BATCH // 16, rowg, 0)
                pltpu.sync_copy(arows, sums_sh.at[didx], add=True)
                pltpu.sync_copy(cval, cnt_sh.at[didx], add=True)

            @pl.when(nb > 0)
            def _():
                issue(0, bufs0)

            def pair(g, carry3):
                k0 = 2 * g

                @pl.when(k0 + 1 < nb)
                def _():
                    issue(k0 + 1, bufs1)

                finish(k0, bufs0)

                @pl.when(k0 + 1 < nb)
                def _():
                    @pl.when(k0 + 2 < nb)
                    def _():
                        issue(k0 + 2, bufs0)

                    finish(k0 + 1, bufs1)

                return carry3

            lax.fori_loop(0, (nb + 1) // 2, pair, 0)
            return carry2

        lax.fori_loop(0, (_K2_NCH - s + _SC_TILES - 1) // _SC_TILES, per_chunk, 0)
        plsc.subcore_barrier()

        nzc = jnp.where(b == _K2_NB - 1, (E - (_K2_NB - 1) * _K2_SEGS) // 32,
                        _K2_SEGS // 32)

        def fl(i, carry2):
            r = (s + i * _SC_TILES) * 32
            pltpu.sync_copy(sums_sh.at[pl.ds(r, 32)],
                            sums_out.at[pl.ds(lo + r, 32)])
            return carry2

        lax.fori_loop(0, (nzc - s + _SC_TILES - 1) // _SC_TILES, fl, 0)
        pltpu.sync_copy(cnt_sh.at[pl.ds(s * zp, zp)], cfl_v.at[pl.ds(0, zp)])
        pltpu.sync_copy(cfl_v.at[pl.ds(0, zp)], cnt_out.at[pl.ds(lo + s * zp, zp)])
        plsc.subcore_barrier()
        return carry

    lax.fori_loop(0, _K2_NB // 2, per_bin, 0)


def _k2(lg_dst, lg_src, lg_att, a_rows, b_rows, w3):
    mesh = plsc.VectorSubcoreMesh(core_axis_name="c", subcore_axis_name="s")
    kfn = pl.kernel(
        _k2_body,
        out_type=[
            jax.ShapeDtypeStruct((_K2_NB * _K2_SEGS, ED), jnp.float32),
            jax.ShapeDtypeStruct((_K2_NB * _K2_SEGS,), jnp.float32),
        ],
        mesh=mesh,
        scratch_types=[
            pltpu.VMEM((_K2_CH,), jnp.int32),
            pltpu.VMEM((_K2_CH,), jnp.int32),
            pltpu.VMEM((_K2_CH,), jnp.float32),
            pltpu.VMEM((_K2_CH + 16,), jnp.int32),
            pltpu.VMEM((_K2_BATCH,), jnp.int32),
            pltpu.VMEM((_K2_BATCH,), jnp.int32),
            pltpu.VMEM((_K2_BATCH,), jnp.int32),
            pltpu.VMEM((_K2_BATCH,), jnp.float32),
            pltpu.VMEM((_K2_BATCH, ED), jnp.float32),
            pltpu.VMEM((_K2_BATCH, ED), jnp.float32),
            pltpu.VMEM((_K2_BATCH,), jnp.int32),
            pltpu.VMEM((_K2_BATCH,), jnp.int32),
            pltpu.VMEM((_K2_BATCH,), jnp.int32),
            pltpu.VMEM((_K2_BATCH,), jnp.float32),
            pltpu.VMEM((_K2_BATCH, ED), jnp.float32),
            pltpu.VMEM((_K2_BATCH, ED), jnp.float32),
            pltpu.VMEM((ED,), jnp.float32),
            pltpu.VMEM((_K2_SEGS // _SC_TILES + 16,), jnp.float32),
            pltpu.VMEM((_K2_SEGS // _SC_TILES + 16,), jnp.float32),
            pltpu.VMEM_SHARED((_K2_SEGS + 8, ED), jnp.float32),
            pltpu.VMEM_SHARED((_K2_SEGS + 8,), jnp.float32),
            pltpu.SemaphoreType.DMA,
            pltpu.SemaphoreType.DMA,
            pltpu.SemaphoreType.DMA,
            pltpu.SemaphoreType.DMA,
            pltpu.SemaphoreType.DMA,
        ],
        compiler_params=pltpu.CompilerParams(needs_layout_passes=False),
    )
    return kfn(lg_dst, lg_src, lg_att, a_rows, b_rows, w3)


_K4_CH = 128


_K4_ZR = 40


def _k4_body(gated_hbm, col_hbm, zeros_hbm, out_hbm, idx_v, rows_v, acc_sh):
    c = lax.axis_index("c")
    s = lax.axis_index("s")
    w = s * _SC_CORES + c
    nzch = N // _K4_ZR

    def zbody(i, carry):
        r = (s + i * _SC_TILES) * _K4_ZR
        pltpu.sync_copy(zeros_hbm, acc_sh.at[pl.ds(r, _K4_ZR)])
        return carry

    nz = (nzch - s + _SC_TILES - 1) // _SC_TILES
    lax.fori_loop(0, nz, zbody, 0)
    plsc.subcore_barrier()

    nch = E // _K4_CH
    nmine = (nch - w + _SC_WORKERS - 1) // _SC_WORKERS

    def body(i, carry):
        off = (w + i * _SC_WORKERS) * _K4_CH
        pltpu.sync_copy(col_hbm.at[pl.ds(off, _K4_CH)], idx_v)
        pltpu.sync_copy(gated_hbm.at[pl.ds(off, _K4_CH), :], rows_v)
        pltpu.sync_copy(rows_v, acc_sh.at[idx_v], add=True)
        return carry

    lax.fori_loop(0, nmine, body, 0)
    plsc.subcore_barrier()

    def fbody(i, carry):
        r = (s + i * _SC_TILES) * _K4_ZR
        pltpu.sync_copy(acc_sh.at[pl.ds(r, _K4_ZR)], out_hbm.at[c, pl.ds(r, _K4_ZR)])
        return carry

    lax.fori_loop(0, nz, fbody, 0)


def _k4(gated, col):
    mesh = plsc.VectorSubcoreMesh(core_axis_name="c", subcore_axis_name="s")
    kfn = pl.kernel(
        _k4_body,
        out_type=jax.ShapeDtypeStruct((_SC_CORES, N, ND), jnp.float32),
        mesh=mesh,
        scratch_types=[
            pltpu.VMEM((_K4_CH,), jnp.int32),
            pltpu.VMEM((_K4_CH, ND), jnp.float32),
            pltpu.VMEM_SHARED((N, ND), jnp.float32),
        ],
    )
    zeros = jnp.zeros((_K4_ZR, ND), jnp.float32)
    return kfn(gated, col, zeros)


def _k5_body(x_ref, aggp_ref, wat_ref, ba_ref, g_ref, bt_ref, out_ref):
    x = x_ref[...]
    agg = aggp_ref[0] + aggp_ref[1]
    z = (jnp.dot(x, wat_ref[:ND, :], preferred_element_type=jnp.float32)
         + jnp.dot(agg, wat_ref[ND:, :], preferred_element_type=jnp.float32)
         + ba_ref[...])
    h = z * jax.nn.sigmoid(z)
    s = x + h
    m = jnp.mean(s, axis=1, keepdims=True)
    v = jnp.mean((s - m) ** 2, axis=1, keepdims=True)
    out_ref[...] = (s - m) / jnp.sqrt(v + _EPS) * g_ref[...] + bt_ref[...]


def _k5(x, agg_parts, wat, b_atom, g_node, bt_node):
    blk = 1000
    grid = N // blk
    return pl.pallas_call(
        _k5_body,
        grid=(grid,),
        in_specs=[
            pl.BlockSpec((blk, ND), lambda i: (i, 0)),
            pl.BlockSpec((2, blk, ND), lambda i: (0, i, 0)),
            pl.BlockSpec((ND + ED, ND), lambda i: (0, 0)),
            pl.BlockSpec((1, ND), lambda i: (0, 0)),
            pl.BlockSpec((1, ND), lambda i: (0, 0)),
            pl.BlockSpec((1, ND), lambda i: (0, 0)),
        ],
        out_specs=pl.BlockSpec((blk, ND), lambda i: (i, 0)),
        out_shape=jax.ShapeDtypeStruct((N, ND), jnp.float32),
    )(x, agg_parts, wat, b_atom.reshape(1, ND), g_node.reshape(1, ND),
      bt_node.reshape(1, ND))


def kernel(x, edge_index, edge_attr, lg_x, lg_edge_index, lg_edge_attr,
           W_line, b_line, W_gate, b_gate, W_atom, b_atom,
           g_node, bt_node, g_edge, bt_edge):
    lg_src = lg_edge_index[0].astype(jnp.int32)
    lg_dst = lg_edge_index[1].astype(jnp.int32)
    col = edge_index[1].astype(jnp.int32)
    lga = lg_edge_attr[:, 0]
    w3 = W_line[:, 2 * ED]

    w12t = jnp.concatenate([W_line[:, :ED].T, W_line[:, ED:2 * ED].T], axis=1)
    a_rows, b_rows = _k1(lg_x, w12t, b_line)

    sums, cnt = _k2(lg_dst, lg_src, lga, a_rows, b_rows, w3)
    cnt2d = cnt.reshape(_K2_NB * _K2_SEGS // 512, 4, 128)

    lg_x_new, ea_new, gated = _k3(edge_attr, lg_x, sums, cnt2d, W_gate.T,
                                  b_gate, g_edge, bt_edge)

    agg_parts = _k4(gated, col)

    x_out = _k5(x, agg_parts, W_atom.T, b_atom, g_node, bt_node)
    return (x_out, ea_new, lg_x_new)


# speedup vs baseline: 1.0420x; 1.0420x over previous
import functools

import jax
import jax.numpy as jnp
from jax import lax
from jax.experimental import pallas as pl
from jax.experimental.pallas import tpu as pltpu
from jax.experimental.pallas import tpu_sc as plsc

N = 10000
E = 320000
E_LG = 640000
ND = 128
ED = 128

_EPS = 1e-5


def _k1_body(lgx_ref, w12t_ref, bl_ref, a_ref, b_ref):
    ab = jnp.dot(lgx_ref[...], w12t_ref[...], preferred_element_type=jnp.float32)
    a_ref[...] = ab[:, :ED]
    b_ref[...] = ab[:, ED:] + bl_ref[...]


def _k1(lg_x, w12t, b_line):
    blk = 512
    grid = E // blk
    return pl.pallas_call(
        _k1_body,
        grid=(grid,),
        in_specs=[
            pl.BlockSpec((blk, ED), lambda i: (i, 0)),
            pl.BlockSpec((ED, 2 * ED), lambda i: (0, 0)),
            pl.BlockSpec((1, ED), lambda i: (0, 0)),
        ],
        out_specs=[
            pl.BlockSpec((blk, ED), lambda i: (i, 0)),
            pl.BlockSpec((blk, ED), lambda i: (i, 0)),
        ],
        out_shape=[
            jax.ShapeDtypeStruct((E, ED), jnp.float32),
            jax.ShapeDtypeStruct((E, ED), jnp.float32),
        ],
    )(lg_x, w12t, b_line.reshape(1, ED))


def _k3_body(ea_ref, lgx_ref, sums_ref, cnt_ref, wgt_ref, bg_ref, g_ref, bt_ref,
             lgxn_ref, out_ea_ref, gated_ref):
    inv4 = 1.0 / jnp.maximum(cnt_ref[0], 1.0)
    invt = inv4.T
    agg = jnp.concatenate(
        [sums_ref[128 * q:128 * (q + 1), :] * invt[:, q:q + 1] for q in range(4)],
        axis=0)
    lgxn = lgx_ref[...] + agg
    lgxn_ref[...] = lgxn
    s = ea_ref[...] + lgxn
    m = jnp.mean(s, axis=1, keepdims=True)
    v = jnp.mean((s - m) ** 2, axis=1, keepdims=True)
    ea = (s - m) / jnp.sqrt(v + _EPS) * g_ref[...] + bt_ref[...]
    gate = jax.nn.sigmoid(
        jnp.dot(ea, wgt_ref[...], preferred_element_type=jnp.float32) + bg_ref[...])
    out_ea_ref[...] = ea
    gated_ref[...] = gate * ea


def _k3(edge_attr, lg_x, sums, cnt2d, wgt, b_gate, g_edge, bt_edge):
    blk = 512
    grid = E // blk
    return pl.pallas_call(
        _k3_body,
        grid=(grid,),
        in_specs=[
            pl.BlockSpec((blk, ED), lambda i: (i, 0)),
            pl.BlockSpec((blk, ED), lambda i: (i, 0)),
            pl.BlockSpec((blk, ED), lambda i: (i, 0)),
            pl.BlockSpec((1, blk // 128, 128), lambda i: (i, 0, 0)),
            pl.BlockSpec((ED, ED), lambda i: (0, 0)),
            pl.BlockSpec((1, ED), lambda i: (0, 0)),
            pl.BlockSpec((1, ED), lambda i: (0, 0)),
            pl.BlockSpec((1, ED), lambda i: (0, 0)),
        ],
        out_specs=[
            pl.BlockSpec((blk, ED), lambda i: (i, 0)),
            pl.BlockSpec((blk, ED), lambda i: (i, 0)),
            pl.BlockSpec((blk, ED), lambda i: (i, 0)),
        ],
        out_shape=[
            jax.ShapeDtypeStruct((E, ED), jnp.float32),
            jax.ShapeDtypeStruct((E, ED), jnp.float32),
            jax.ShapeDtypeStruct((E, ED), jnp.float32),
        ],
    )(edge_attr, lg_x, sums, cnt2d, wgt, b_gate.reshape(1, ED),
      g_edge.reshape(1, ED), bt_edge.reshape(1, ED))


_SC_CORES = 2
_SC_TILES = 16
_SC_WORKERS = _SC_CORES * _SC_TILES


_K2_NB = 40
_K2_SEGS = 8064
_K2_CH = 5120
_K2_NCH = E_LG // _K2_CH
_K2_BATCH = 64


def _k2_body(dst_hbm, src_hbm, att_hbm, a_hbm, b_hbm, w3_hbm,
             sums_out, cnt_out,
             dst_ch, src_ch, att_ch, pos_c,
             sidx_a, gdst_a, didx_a, cval_a, arows_a, brows_a,
             sidx_b, gdst_b, didx_b, cval_b, arows_b, brows_b,
             w3_vm, zcnt_v, cfl_v,
             sums_sh, cnt_sh, semd, sema1, semb1, sema2, semb2, semsa, semsb):
    c = lax.axis_index("c")
    s = lax.axis_index("s")
    zp = _K2_SEGS // _SC_TILES

    pltpu.sync_copy(w3_hbm, w3_vm)
    w3v = [w3_vm[pl.ds(16 * v, 16)] for v in range(8)]

    zv16 = jnp.zeros((16,), jnp.float32)
    zv16i = jnp.zeros((16,), jnp.int32)

    def zc(i, carry):
        zcnt_v[pl.ds(16 * i, 16)] = zv16
        return carry

    lax.fori_loop(0, (zp + 15) // 16, zc, 0)

    def zg(i, carry):
        pos_c[pl.ds(16 * i, 16)] = zv16i
        return carry

    lax.fori_loop(0, (_K2_CH + 16) // 16, zg, 0)

    bufs0 = (sidx_a, gdst_a, didx_a, cval_a, arows_a, brows_a, sema1, sema2, semsa)
    bufs1 = (sidx_b, gdst_b, didx_b, cval_b, arows_b, brows_b, semb1, semb2, semsb)

    def per_bin(bi, carry):
        b = c * (_K2_NB // 2) + bi
        lo = b * _K2_SEGS

        def zr(i, carry2):
            for v in range(8):
                arows_a[i, pl.ds(16 * v, 16)] = zv16
            return carry2

        lax.fori_loop(0, _K2_BATCH, zr, 0)

        def z(i, carry2):
            r = (s + i * _SC_TILES) * _K2_BATCH
            pltpu.sync_copy(arows_a, sums_sh.at[pl.ds(r, _K2_BATCH)])
            return carry2

        lax.fori_loop(0, ((_K2_SEGS // _K2_BATCH) - s + _SC_TILES - 1) // _SC_TILES,
                      z, 0)
        pltpu.sync_copy(zcnt_v.at[pl.ds(0, zp)], cnt_sh.at[pl.ds(s * zp, zp)])
        plsc.subcore_barrier()

        def per_chunk(ic, carry2):
            cid = s + ic * _SC_TILES
            base_g = cid * _K2_CH
            cpd = pltpu.async_copy(dst_hbm.at[pl.ds(base_g, _K2_CH)], dst_ch, semd)
            cps = pltpu.async_copy(src_hbm.at[pl.ds(base_g, _K2_CH)], src_ch, sema1)
            cpa = pltpu.async_copy(att_hbm.at[pl.ds(base_g, _K2_CH)], att_ch, semb1)
            cpd.wait()

            def sc(j, nacc):
                d = dst_ch[pl.ds(16 * j, 16)]
                m = (d >= lo) & (d < lo + _K2_SEGS)
                posv = 16 * j + lax.iota(jnp.int32, 16)
                plsc.store_compressed(pos_c.at[pl.ds(nacc, 16)], posv, mask=m)
                pc = plsc.all_reduce_population_count(m)
                return nacc + pc[0]

            nacc = lax.fori_loop(0, _K2_CH // 16, sc, 0)
            cps.wait()
            cpa.wait()
            nb = (nacc + _K2_BATCH - 1) // _K2_BATCH

            def drain_scatters(bufs):
                sidx, gdst, didx, cval, arows, brows, s1, s2, ss = bufs
                pltpu.make_async_copy(arows, sums_sh.at[didx], ss).wait()
                pltpu.make_async_copy(cval, cnt_sh.at[didx], ss).wait()

            def issue(k, bufs):
                sidx, gdst, didx, cval, arows, brows, s1, s2, ss = bufs
                kb = k * _K2_BATCH

                @pl.when(k >= 2)
                def _():
                    drain_scatters(bufs)
                for v in range(_K2_BATCH // 16):
                    pos = pos_c[pl.ds(kb + 16 * v, 16)]
                    dv = plsc.load_gather(dst_ch, [pos])
                    sv = plsc.load_gather(src_ch, [pos])
                    valid = (kb + 16 * v + lax.iota(jnp.int32, 16)) < nacc
                    sidx[pl.ds(16 * v, 16)] = sv
                    gdst[pl.ds(16 * v, 16)] = dv
                    didx[pl.ds(16 * v, 16)] = jnp.where(valid, dv - lo, _K2_SEGS)
                    cval[pl.ds(16 * v, 16)] = jnp.where(valid, 1.0, 0.0)
                pltpu.async_copy(a_hbm.at[sidx], arows, s1)
                pltpu.async_copy(b_hbm.at[gdst], brows, s2)

            def finish(k, bufs):
                sidx, gdst, didx, cval, arows, brows, s1, s2, ss = bufs
                kb = k * _K2_BATCH
                pltpu.make_async_copy(a_hbm.at[sidx], arows, s1).wait()
                pltpu.make_async_copy(b_hbm.at[gdst], brows, s2).wait()

                def rowg(g, carry4):
                    attv = plsc.load_gather(att_ch, [pos_c[pl.ds(kb + 16 * g, 16)]])
                    for r2 in range(16):
                        att = attv[r2]
                        r = 16 * g + r2
                        for v in range(8):
                            av = arows[r, pl.ds(16 * v, 16)]
                            bv = brows[r, pl.ds(16 * v, 16)]
                            zv = av + bv + att * w3v[v]
                            arows[r, pl.ds(16 * v, 16)] = zv / (1.0 + jnp.exp(-zv))
                    return carry4

                lax.fori_loop(0, _K2_BATCH // 16, rowg, 0)
                pltpu.async_copy(arows, sums_sh.at[didx], ss, add=True)
                pltpu.async_copy(cval, cnt_sh.at[didx], ss, add=True)

            @pl.when(nb > 0)
            def _():
                issue(0, bufs0)

            def pair(g, carry3):
                k0 = 2 * g

                @pl.when(k0 + 1 < nb)
                def _():
                    issue(k0 + 1, bufs1)

                finish(k0, bufs0)

                @pl.when(k0 + 1 < nb)
                def _():
                    @pl.when(k0 + 2 < nb)
                    def _():
                        issue(k0 + 2, bufs0)

                    finish(k0 + 1, bufs1)

                return carry3

            lax.fori_loop(0, (nb + 1) // 2, pair, 0)

            @pl.when(nb >= 1)
            def _():
                drain_scatters(bufs0)

            @pl.when(nb >= 2)
            def _():
                drain_scatters(bufs1)

            return carry2

        lax.fori_loop(0, (_K2_NCH - s + _SC_TILES - 1) // _SC_TILES, per_chunk, 0)
        plsc.subcore_barrier()

        nzc = jnp.where(b == _K2_NB - 1, (E - (_K2_NB - 1) * _K2_SEGS) // 32,
                        _K2_SEGS // 32)

        def fl(i, carry2):
            r = (s + i * _SC_TILES) * 32
            pltpu.sync_copy(sums_sh.at[pl.ds(r, 32)],
                            sums_out.at[pl.ds(lo + r, 32)])
            return carry2

        lax.fori_loop(0, (nzc - s + _SC_TILES - 1) // _SC_TILES, fl, 0)
        pltpu.sync_copy(cnt_sh.at[pl.ds(s * zp, zp)], cfl_v.at[pl.ds(0, zp)])
        pltpu.sync_copy(cfl_v.at[pl.ds(0, zp)], cnt_out.at[pl.ds(lo + s * zp, zp)])
        plsc.subcore_barrier()
        return carry

    lax.fori_loop(0, _K2_NB // 2, per_bin, 0)


def _k2(lg_dst, lg_src, lg_att, a_rows, b_rows, w3):
    mesh = plsc.VectorSubcoreMesh(core_axis_name="c", subcore_axis_name="s")
    kfn = pl.kernel(
        _k2_body,
        out_type=[
            jax.ShapeDtypeStruct((_K2_NB * _K2_SEGS, ED), jnp.float32),
            jax.ShapeDtypeStruct((_K2_NB * _K2_SEGS,), jnp.float32),
        ],
        mesh=mesh,
        scratch_types=[
            pltpu.VMEM((_K2_CH,), jnp.int32),
            pltpu.VMEM((_K2_CH,), jnp.int32),
            pltpu.VMEM((_K2_CH,), jnp.float32),
            pltpu.VMEM((_K2_CH + 16,), jnp.int32),
            pltpu.VMEM((_K2_BATCH,), jnp.int32),
            pltpu.VMEM((_K2_BATCH,), jnp.int32),
            pltpu.VMEM((_K2_BATCH,), jnp.int32),
            pltpu.VMEM((_K2_BATCH,), jnp.float32),
            pltpu.VMEM((_K2_BATCH, ED), jnp.float32),
            pltpu.VMEM((_K2_BATCH, ED), jnp.float32),
            pltpu.VMEM((_K2_BATCH,), jnp.int32),
            pltpu.VMEM((_K2_BATCH,), jnp.int32),
            pltpu.VMEM((_K2_BATCH,), jnp.int32),
            pltpu.VMEM((_K2_BATCH,), jnp.float32),
            pltpu.VMEM((_K2_BATCH, ED), jnp.float32),
            pltpu.VMEM((_K2_BATCH, ED), jnp.float32),
            pltpu.VMEM((ED,), jnp.float32),
            pltpu.VMEM((_K2_SEGS // _SC_TILES + 16,), jnp.float32),
            pltpu.VMEM((_K2_SEGS // _SC_TILES + 16,), jnp.float32),
            pltpu.VMEM_SHARED((_K2_SEGS + 8, ED), jnp.float32),
            pltpu.VMEM_SHARED((_K2_SEGS + 8,), jnp.float32),
            pltpu.SemaphoreType.DMA,
            pltpu.SemaphoreType.DMA,
            pltpu.SemaphoreType.DMA,
            pltpu.SemaphoreType.DMA,
            pltpu.SemaphoreType.DMA,
            pltpu.SemaphoreType.DMA,
            pltpu.SemaphoreType.DMA,
        ],
        compiler_params=pltpu.CompilerParams(needs_layout_passes=False),
    )
    return kfn(lg_dst, lg_src, lg_att, a_rows, b_rows, w3)


_K4_CH = 128


_K4_ZR = 40


def _k4_body(gated_hbm, col_hbm, zeros_hbm, out_hbm, idx_v, rows_v, acc_sh):
    c = lax.axis_index("c")
    s = lax.axis_index("s")
    w = s * _SC_CORES + c
    nzch = N // _K4_ZR

    def zbody(i, carry):
        r = (s + i * _SC_TILES) * _K4_ZR
        pltpu.sync_copy(zeros_hbm, acc_sh.at[pl.ds(r, _K4_ZR)])
        return carry

    nz = (nzch - s + _SC_TILES - 1) // _SC_TILES
    lax.fori_loop(0, nz, zbody, 0)
    plsc.subcore_barrier()

    nch = E // _K4_CH
    nmine = (nch - w + _SC_WORKERS - 1) // _SC_WORKERS

    def body(i, carry):
        off = (w + i * _SC_WORKERS) * _K4_CH
        pltpu.sync_copy(col_hbm.at[pl.ds(off, _K4_CH)], idx_v)
        pltpu.sync_copy(gated_hbm.at[pl.ds(off, _K4_CH), :], rows_v)
        pltpu.sync_copy(rows_v, acc_sh.at[idx_v], add=True)
        return carry

    lax.fori_loop(0, nmine, body, 0)
    plsc.subcore_barrier()

    def fbody(i, carry):
        r = (s + i * _SC_TILES) * _K4_ZR
        pltpu.sync_copy(acc_sh.at[pl.ds(r, _K4_ZR)], out_hbm.at[c, pl.ds(r, _K4_ZR)])
        return carry

    lax.fori_loop(0, nz, fbody, 0)


def _k4(gated, col):
    mesh = plsc.VectorSubcoreMesh(core_axis_name="c", subcore_axis_name="s")
    kfn = pl.kernel(
        _k4_body,
        out_type=jax.ShapeDtypeStruct((_SC_CORES, N, ND), jnp.float32),
        mesh=mesh,
        scratch_types=[
            pltpu.VMEM((_K4_CH,), jnp.int32),
            pltpu.VMEM((_K4_CH, ND), jnp.float32),
            pltpu.VMEM_SHARED((N, ND), jnp.float32),
        ],
    )
    zeros = jnp.zeros((_K4_ZR, ND), jnp.float32)
    return kfn(gated, col, zeros)


def _k5_body(x_ref, aggp_ref, wat_ref, ba_ref, g_ref, bt_ref, out_ref):
    x = x_ref[...]
    agg = aggp_ref[0] + aggp_ref[1]
    z = (jnp.dot(x, wat_ref[:ND, :], preferred_element_type=jnp.float32)
         + jnp.dot(agg, wat_ref[ND:, :], preferred_element_type=jnp.float32)
         + ba_ref[...])
    h = z * jax.nn.sigmoid(z)
    s = x + h
    m = jnp.mean(s, axis=1, keepdims=True)
    v = jnp.mean((s - m) ** 2, axis=1, keepdims=True)
    out_ref[...] = (s - m) / jnp.sqrt(v + _EPS) * g_ref[...] + bt_ref[...]


def _k5(x, agg_parts, wat, b_atom, g_node, bt_node):
    blk = 1000
    grid = N // blk
    return pl.pallas_call(
        _k5_body,
        grid=(grid,),
        in_specs=[
            pl.BlockSpec((blk, ND), lambda i: (i, 0)),
            pl.BlockSpec((2, blk, ND), lambda i: (0, i, 0)),
            pl.BlockSpec((ND + ED, ND), lambda i: (0, 0)),
            pl.BlockSpec((1, ND), lambda i: (0, 0)),
            pl.BlockSpec((1, ND), lambda i: (0, 0)),
            pl.BlockSpec((1, ND), lambda i: (0, 0)),
        ],
        out_specs=pl.BlockSpec((blk, ND), lambda i: (i, 0)),
        out_shape=jax.ShapeDtypeStruct((N, ND), jnp.float32),
    )(x, agg_parts, wat, b_atom.reshape(1, ND), g_node.reshape(1, ND),
      bt_node.reshape(1, ND))


def kernel(x, edge_index, edge_attr, lg_x, lg_edge_index, lg_edge_attr,
           W_line, b_line, W_gate, b_gate, W_atom, b_atom,
           g_node, bt_node, g_edge, bt_edge):
    lg_src = lg_edge_index[0].astype(jnp.int32)
    lg_dst = lg_edge_index[1].astype(jnp.int32)
    col = edge_index[1].astype(jnp.int32)
    lga = lg_edge_attr[:, 0]
    w3 = W_line[:, 2 * ED]

    w12t = jnp.concatenate([W_line[:, :ED].T, W_line[:, ED:2 * ED].T], axis=1)
    a_rows, b_rows = _k1(lg_x, w12t, b_line)

    sums, cnt = _k2(lg_dst, lg_src, lga, a_rows, b_rows, w3)
    cnt2d = cnt.reshape(_K2_NB * _K2_SEGS // 512, 4, 128)

    lg_x_new, ea_new, gated = _k3(edge_attr, lg_x, sums, cnt2d, W_gate.T,
                                  b_gate, g_edge, bt_edge)

    agg_parts = _k4(gated, col)

    x_out = _k5(x, agg_parts, W_atom.T, b_atom, g_node, bt_node)
    return (x_out, ea_new, lg_x_new)
